# Initial kernel scaffold; baseline (speedup 1.0000x reference)
import functools

import jax
import jax.numpy as jnp
from jax import lax
from jax.experimental import pallas as pl
from jax.experimental.pallas import tpu as pltpu
from jax.experimental.pallas import tpu_sc as plsc

N = 10000
E = 320000
D = 128

NC = 2
NS = 16
LANES = 16
NW = NC * NS

NPAD = 10240
B = 128
NB_TILE = 80
EPAD = NW * NB_TILE * B
ROWS_PER_TILE = NPAD // NS

_HIGHEST = jax.lax.Precision.HIGHEST


def _dot(a, b):
  return jax.lax.dot(a, b, precision=_HIGHEST,
                     preferred_element_type=jnp.float32)


def _lrelu(v):
  return jnp.where(v >= 0, v, v * jnp.float32(0.2))


def _prep_body(x_ref, ws_ref, wd_ref, avs_ref, avd_ref, wl_ref, bl_ref,
               xs_ref, asrc_ref, ad_ref, u_ref, skip_ref):
  x = x_ref[...]
  xs = _dot(x, ws_ref[...])
  xd = _dot(x, wd_ref[...])
  xs_ref[...] = xs
  asrc = jnp.sum(xs * avs_ref[...], axis=1, keepdims=True)
  ad = jnp.sum(xd * avd_ref[...], axis=1, keepdims=True)
  asrc_ref[...] = asrc
  ad_ref[...] = ad
  m = jnp.max(asrc)
  u_ref[...] = _lrelu(m + ad)
  skip_ref[...] = _dot(x, wl_ref[...]) + bl_ref[...]


def _tc_prep(xp, Ws, Wd, avs, avd, Wl, bl):
  out_shape = (
      jax.ShapeDtypeStruct((NPAD, D), jnp.float32),
      jax.ShapeDtypeStruct((NPAD, 1), jnp.float32),
      jax.ShapeDtypeStruct((NPAD, 1), jnp.float32),
      jax.ShapeDtypeStruct((NPAD, 1), jnp.float32),
      jax.ShapeDtypeStruct((NPAD, D), jnp.float32),
  )
  return pl.pallas_call(_prep_body, out_shape=out_shape)(
      xp, Ws, Wd, avs, avd, Wl, bl)


def _mid_body(n_ref, d_ref, b1_ref, skip1_ref, ws_ref, wd_ref, avs_ref,
              avd_ref, wl_ref, bl_ref,
              xs_ref, asrc_ref, ad_ref, u_ref, skip_ref):
  dsum = d_ref[0] + d_ref[1]
  gat = (n_ref[0] + n_ref[1]) / (dsum + jnp.float32(1e-16)) + b1_ref[...]
  h = jnp.maximum(gat + skip1_ref[...], 0.0)
  rowid = jax.lax.broadcasted_iota(jnp.int32, (NPAD, 1), 0)
  h = jnp.where(rowid < N, h, 0.0)
  xs = _dot(h, ws_ref[...])
  xd = _dot(h, wd_ref[...])
  xs_ref[...] = xs
  asrc = jnp.sum(xs * avs_ref[...], axis=1, keepdims=True)
  ad = jnp.sum(xd * avd_ref[...], axis=1, keepdims=True)
  asrc_ref[...] = asrc
  ad_ref[...] = ad
  m = jnp.max(asrc)
  u_ref[...] = _lrelu(m + ad)
  skip_ref[...] = _dot(h, wl_ref[...]) + bl_ref[...]


def _tc_mid(numer, denom, b1, skip1, Ws, Wd, avs, avd, Wl, bl):
  out_shape = (
      jax.ShapeDtypeStruct((NPAD, D), jnp.float32),
      jax.ShapeDtypeStruct((NPAD, 1), jnp.float32),
      jax.ShapeDtypeStruct((NPAD, 1), jnp.float32),
      jax.ShapeDtypeStruct((NPAD, 1), jnp.float32),
      jax.ShapeDtypeStruct((NPAD, D), jnp.float32),
  )
  return pl.pallas_call(_mid_body, out_shape=out_shape)(
      numer, denom, b1, skip1, Ws, Wd, avs, avd, Wl, bl)


def _final_body(n_ref, d_ref, b2_ref, skip2_ref, out_ref):
  dsum = d_ref[0] + d_ref[1]
  gat = (n_ref[0] + n_ref[1]) / (dsum + jnp.float32(1e-16)) + b2_ref[...]
  out_ref[...] = gat + skip2_ref[...]


def _tc_final(numer, denom, b2, skip2):
  return pl.pallas_call(
      _final_body,
      out_shape=jax.ShapeDtypeStruct((NPAD, D), jnp.float32),
  )(numer, denom, b2, skip2)


_SC_MESH = plsc.VectorSubcoreMesh(
    core_axis_name="c", subcore_axis_name="s", num_cores=NC, num_subcores=NS)


@functools.partial(
    pl.kernel,
    out_type=(
        jax.ShapeDtypeStruct((NC, NPAD, D), jnp.float32),
        jax.ShapeDtypeStruct((NC, NPAD), jnp.float32),
    ),
    mesh=_SC_MESH,
    scratch_types=[
        pltpu.VMEM((NB_TILE, B), jnp.int32),
        pltpu.VMEM((NB_TILE, B), jnp.int32),
        pltpu.VMEM((NPAD,), jnp.float32),
        pltpu.VMEM((NPAD,), jnp.float32),
        pltpu.VMEM((NPAD,), jnp.float32),
        pltpu.VMEM((B, D), jnp.float32),
        pltpu.VMEM((B,), jnp.float32),
        pltpu.VMEM_SHARED((NPAD, D), jnp.float32),
        pltpu.VMEM_SHARED((NPAD,), jnp.float32),
        pltpu.SemaphoreType.DMA,
    ],
)
def _sc_edge_kernel(xs_hbm, asrc_hbm, ad_hbm, u_hbm, src_hbm, dst_hbm,
                    zr_hbm, zv_hbm, numer_hbm, denom_hbm,
                    idxs_v, idxd_v, tas_v, tad_v, tu_v, rows_v, ex_v,
                    sh_numer, sh_denom, sem):
  c = lax.axis_index("c")
  s = lax.axis_index("s")
  wid = c * NS + s
  r0 = s * ROWS_PER_TILE

  pltpu.sync_copy(zr_hbm, sh_numer.at[pl.ds(r0, ROWS_PER_TILE)])
  pltpu.sync_copy(zv_hbm, sh_denom.at[pl.ds(r0, ROWS_PER_TILE)])

  pltpu.sync_copy(asrc_hbm, tas_v)
  pltpu.sync_copy(ad_hbm, tad_v)
  pltpu.sync_copy(u_hbm, tu_v)
  pltpu.sync_copy(src_hbm.at[pl.ds(wid * NB_TILE, NB_TILE)], idxs_v)
  pltpu.sync_copy(dst_hbm.at[pl.ds(wid * NB_TILE, NB_TILE)], idxd_v)
  plsc.subcore_barrier()

  @pl.loop(0, NB_TILE)
  def _batch(g):
    gath = pltpu.async_copy(xs_hbm.at[idxs_v.at[g]], rows_v, sem)
    for k in range(B // LANES):
      sl = pl.ds(k * LANES, LANES)
      sv = idxs_v[g, sl]
      dv = idxd_v[g, sl]
      a = plsc.load_gather(tas_v, [sv]) + plsc.load_gather(tad_v, [dv])
      ex = jnp.exp(_lrelu(a) - plsc.load_gather(tu_v, [dv]))
      ex_v[sl] = ex
    gath.wait()

    @pl.loop(0, B)
    def _row(r):
      ev = jnp.full((LANES,), ex_v[r], jnp.float32)
      for j in range(D // LANES):
        sl = pl.ds(j * LANES, LANES)
        rows_v[r, sl] = rows_v[r, sl] * ev

    pltpu.sync_copy(rows_v, sh_numer.at[idxd_v.at[g]], add=True)
    pltpu.sync_copy(ex_v, sh_denom.at[idxd_v.at[g]], add=True)

  plsc.subcore_barrier()
  pltpu.sync_copy(sh_numer.at[pl.ds(r0, ROWS_PER_TILE)],
                  numer_hbm.at[c, pl.ds(r0, ROWS_PER_TILE)])
  pltpu.sync_copy(sh_denom.at[pl.ds(r0, ROWS_PER_TILE)],
                  denom_hbm.at[c, pl.ds(r0, ROWS_PER_TILE)])


def kernel(x, edge_index, W1s, W1d, a1s, a1d, b1, Wl1, bl1,
           W2s, W2d, a2s, a2d, b2, Wl2, bl2):
  src = edge_index[0].astype(jnp.int32)
  dst = edge_index[1].astype(jnp.int32)
  pad = jnp.full((EPAD - E,), N, jnp.int32)
  srcp = jnp.concatenate([src, pad]).reshape(NW * NB_TILE, B)
  dstp = jnp.concatenate([dst, pad]).reshape(NW * NB_TILE, B)

  xp = jnp.zeros((NPAD, D), jnp.float32).at[:N].set(x)
  zr = jnp.zeros((ROWS_PER_TILE, D), jnp.float32)
  zv = jnp.zeros((ROWS_PER_TILE,), jnp.float32)

  a1s_v = a1s.reshape(1, D)
  a1d_v = a1d.reshape(1, D)
  a2s_v = a2s.reshape(1, D)
  a2d_v = a2d.reshape(1, D)

  xs1, asrc1, ad1, u1, skip1 = _tc_prep(
      xp, W1s, W1d, a1s_v, a1d_v, Wl1, bl1.reshape(1, D))
  numer1, denom1 = _sc_edge_kernel(
      xs1, asrc1.reshape(NPAD), ad1.reshape(NPAD), u1.reshape(NPAD),
      srcp, dstp, zr, zv)

  xs2, asrc2, ad2, u2, skip2 = _tc_mid(
      numer1, denom1.reshape(NC, NPAD, 1), b1.reshape(1, D), skip1,
      W2s, W2d, a2s_v, a2d_v, Wl2, bl2.reshape(1, D))
  numer2, denom2 = _sc_edge_kernel(
      xs2, asrc2.reshape(NPAD), ad2.reshape(NPAD), u2.reshape(NPAD),
      srcp, dstp, zr, zv)

  out = _tc_final(numer2, denom2.reshape(NC, NPAD, 1), b2.reshape(1, D),
                  skip2)
  return out[:N]


# speedup vs baseline: 15.3312x; 15.3312x over previous
import dataclasses
import functools

import jax
import jax.numpy as jnp
from jax import lax
from jax.experimental import pallas as pl
from jax.experimental.pallas import tpu as pltpu
from jax.experimental.pallas import tpu_sc as plsc

N = 10000
E = 320000
D = 128

NC = 2
NS = 16
LANES = 16
NW = NC * NS

DH = D // 2
NPAD = 10240
B = 128
NB_TILE = 160
EPAD = NS * NB_TILE * B
ROWS_PER_TILE = NPAD // NS

_HIGHEST = jax.lax.Precision.HIGHEST


def _dot(a, b):
  return jax.lax.dot(a, b, precision=_HIGHEST,
                     preferred_element_type=jnp.float32)


def _lrelu(v):
  return jnp.where(v >= 0, v, v * jnp.float32(0.2))


BLK = 2048
GRID = NPAD // BLK

_row_spec = pl.BlockSpec((BLK, D), lambda i: (i, 0))
_col_spec = pl.BlockSpec((BLK, 1), lambda i: (i, 0))
_xs_spec = pl.BlockSpec((NC, BLK, DH), lambda i: (0, i, 0))
_w_spec = pl.BlockSpec((D, D), lambda i: (0, 0))
_v_spec = pl.BlockSpec((1, D), lambda i: (0, 0))


def _prep_body(x_ref, ws_ref, wd_ref, avs_ref, avd_ref, wl_ref, bl_ref,
               xs_ref, asrc_ref, ad_ref, skip_ref):
  x = x_ref[...]
  xs = _dot(x, ws_ref[...])
  xd = _dot(x, wd_ref[...])
  xs_ref[0] = xs[:, :DH]
  xs_ref[1] = xs[:, DH:]
  asrc_ref[...] = jnp.sum(xs * avs_ref[...], axis=1, keepdims=True)
  ad_ref[...] = jnp.sum(xd * avd_ref[...], axis=1, keepdims=True)
  skip_ref[...] = _dot(x, wl_ref[...]) + bl_ref[...]


def _tc_prep(xp, Ws, Wd, avs, avd, Wl, bl):
  out_shape = (
      jax.ShapeDtypeStruct((NC, NPAD, DH), jnp.float32),
      jax.ShapeDtypeStruct((NPAD, 1), jnp.float32),
      jax.ShapeDtypeStruct((NPAD, 1), jnp.float32),
      jax.ShapeDtypeStruct((NPAD, D), jnp.float32),
  )
  return pl.pallas_call(
      _prep_body,
      grid=(GRID,),
      in_specs=[_row_spec, _w_spec, _w_spec, _v_spec, _v_spec, _w_spec,
                _v_spec],
      out_specs=(_xs_spec, _col_spec, _col_spec, _row_spec),
      out_shape=out_shape,
  )(xp, Ws, Wd, avs, avd, Wl, bl)


def _u_body(asrc_ref, ad_ref, u_ref):
  m = jnp.max(asrc_ref[...])
  u_ref[...] = _lrelu(m + ad_ref[...])


def _tc_u(asrc, ad):
  return pl.pallas_call(
      _u_body,
      out_shape=jax.ShapeDtypeStruct((NPAD, 1), jnp.float32),
  )(asrc, ad)


def _gat_h(n_ref, d_ref, b_ref, skip_ref):
  numer = jnp.concatenate([n_ref[0], n_ref[1]], axis=1)
  return numer / (d_ref[...] + jnp.float32(1e-16)) + b_ref[...] + skip_ref[...]


def _mid_body(n_ref, d_ref, b1_ref, skip1_ref, ws_ref, wd_ref, avs_ref,
              avd_ref, wl_ref, bl_ref,
              xs_ref, asrc_ref, ad_ref, skip_ref):
  h = jnp.maximum(_gat_h(n_ref, d_ref, b1_ref, skip1_ref), 0.0)
  base = pl.program_id(0) * BLK
  rowid = base + jax.lax.broadcasted_iota(jnp.int32, (BLK, 1), 0)
  h = jnp.where(rowid < N, h, 0.0)
  xs = _dot(h, ws_ref[...])
  xd = _dot(h, wd_ref[...])
  xs_ref[0] = xs[:, :DH]
  xs_ref[1] = xs[:, DH:]
  asrc_ref[...] = jnp.sum(xs * avs_ref[...], axis=1, keepdims=True)
  ad_ref[...] = jnp.sum(xd * avd_ref[...], axis=1, keepdims=True)
  skip_ref[...] = _dot(h, wl_ref[...]) + bl_ref[...]


def _tc_mid(numer, denom, b1, skip1, Ws, Wd, avs, avd, Wl, bl):
  out_shape = (
      jax.ShapeDtypeStruct((NC, NPAD, DH), jnp.float32),
      jax.ShapeDtypeStruct((NPAD, 1), jnp.float32),
      jax.ShapeDtypeStruct((NPAD, 1), jnp.float32),
      jax.ShapeDtypeStruct((NPAD, D), jnp.float32),
  )
  return pl.pallas_call(
      _mid_body,
      grid=(GRID,),
      in_specs=[_xs_spec, _col_spec, _v_spec, _row_spec, _w_spec, _w_spec,
                _v_spec, _v_spec, _w_spec, _v_spec],
      out_specs=(_xs_spec, _col_spec, _col_spec, _row_spec),
      out_shape=out_shape,
  )(numer, denom, b1, skip1, Ws, Wd, avs, avd, Wl, bl)


def _final_body(n_ref, d_ref, b2_ref, skip2_ref, out_ref):
  out_ref[...] = _gat_h(n_ref, d_ref, b2_ref, skip2_ref)


def _tc_final(numer, denom, b2, skip2):
  return pl.pallas_call(
      _final_body,
      grid=(GRID,),
      in_specs=[_xs_spec, _col_spec, _v_spec, _row_spec],
      out_specs=_row_spec,
      out_shape=jax.ShapeDtypeStruct((NPAD, D), jnp.float32),
  )(numer, denom, b2, skip2)


@functools.cache
def _make_sc_edge_kernel():
  mesh = plsc.VectorSubcoreMesh(
      core_axis_name="c", subcore_axis_name="s",
      num_cores=NC, num_subcores=NS)

  cp = pltpu.CompilerParams()
  if "needs_layout_passes" in pltpu.CompilerParams.__dataclass_fields__:
    cp = dataclasses.replace(cp, needs_layout_passes=False)
  if "use_tc_tiling_on_sc" in pltpu.CompilerParams.__dataclass_fields__:
    cp = dataclasses.replace(cp, use_tc_tiling_on_sc=False)

  @functools.partial(
      pl.kernel,
      compiler_params=cp,
      out_type=(
          jax.ShapeDtypeStruct((NC, NPAD, DH), jnp.float32),
          jax.ShapeDtypeStruct((NC, NPAD), jnp.float32),
      ),
      mesh=mesh,
      scratch_types=[
          pltpu.VMEM((NB_TILE, B), jnp.int32),
          pltpu.VMEM((NB_TILE, B), jnp.int32),
          pltpu.VMEM((NPAD,), jnp.float32),
          pltpu.VMEM((NPAD,), jnp.float32),
          pltpu.VMEM((NPAD,), jnp.float32),
          pltpu.VMEM((B, DH), jnp.float32),
          pltpu.VMEM((B,), jnp.float32),
          pltpu.VMEM_SHARED((NPAD, DH), jnp.float32),
          pltpu.VMEM_SHARED((NPAD,), jnp.float32),
          pltpu.SemaphoreType.DMA,
      ],
  )
  def _sc_edge_kernel(xs_hbm, asrc_hbm, ad_hbm, u_hbm, src_hbm, dst_hbm,
                      zr_hbm, zv_hbm, numer_hbm, denom_hbm,
                      idxs_v, idxd_v, tas_v, tad_v, tu_v, rows_v, ex_v,
                      sh_numer, sh_denom, sem):
    _sc_edge_body(xs_hbm, asrc_hbm, ad_hbm, u_hbm, src_hbm, dst_hbm,
                  zr_hbm, zv_hbm, numer_hbm, denom_hbm,
                  idxs_v, idxd_v, tas_v, tad_v, tu_v, rows_v, ex_v,
                  sh_numer, sh_denom, sem)

  return _sc_edge_kernel


def _sc_edge_body(xs_hbm, asrc_hbm, ad_hbm, u_hbm, src_hbm, dst_hbm,
                  zr_hbm, zv_hbm, numer_hbm, denom_hbm,
                  idxs_v, idxd_v, tas_v, tad_v, tu_v, rows_v, ex_v,
                  sh_numer, sh_denom, sem):
  c = lax.axis_index("c")
  s = lax.axis_index("s")
  r0 = s * ROWS_PER_TILE
  xs_half = xs_hbm.at[c]

  pltpu.sync_copy(zr_hbm, sh_numer.at[pl.ds(r0, ROWS_PER_TILE)])
  pltpu.sync_copy(zv_hbm, sh_denom.at[pl.ds(r0, ROWS_PER_TILE)])

  pltpu.sync_copy(asrc_hbm, tas_v)
  pltpu.sync_copy(ad_hbm, tad_v)
  pltpu.sync_copy(u_hbm, tu_v)
  pltpu.sync_copy(src_hbm.at[pl.ds(s * NB_TILE, NB_TILE)], idxs_v)
  pltpu.sync_copy(dst_hbm.at[pl.ds(s * NB_TILE, NB_TILE)], idxd_v)
  plsc.subcore_barrier()

  @pl.loop(0, NB_TILE)
  def _batch(g):
    gath = pltpu.async_copy(xs_half.at[idxs_v.at[g]], rows_v, sem)
    for k in range(B // LANES):
      sl = pl.ds(k * LANES, LANES)
      sv = idxs_v[g, sl]
      dv = idxd_v[g, sl]
      a = plsc.load_gather(tas_v, [sv]) + plsc.load_gather(tad_v, [dv])
      ex = jnp.exp(_lrelu(a) - plsc.load_gather(tu_v, [dv]))
      ex_v[sl] = ex
    gath.wait()

    @pl.loop(0, B)
    def _row(r):
      ev = plsc.load_gather(ex_v, [jnp.full((LANES,), r, jnp.int32)])
      for j in range(DH // LANES):
        sl = pl.ds(j * LANES, LANES)
        rows_v[r, sl] = rows_v[r, sl] * ev

    pltpu.sync_copy(rows_v, sh_numer.at[idxd_v.at[g]], add=True)
    pltpu.sync_copy(ex_v, sh_denom.at[idxd_v.at[g]], add=True)

  plsc.subcore_barrier()
  pltpu.sync_copy(sh_numer.at[pl.ds(r0, ROWS_PER_TILE)],
                  numer_hbm.at[c, pl.ds(r0, ROWS_PER_TILE)])
  pltpu.sync_copy(sh_denom.at[pl.ds(r0, ROWS_PER_TILE)],
                  denom_hbm.at[c, pl.ds(r0, ROWS_PER_TILE)])


def kernel(x, edge_index, W1s, W1d, a1s, a1d, b1, Wl1, bl1,
           W2s, W2d, a2s, a2d, b2, Wl2, bl2):
  src = edge_index[0].astype(jnp.int32)
  dst = edge_index[1].astype(jnp.int32)
  pad = jnp.full((EPAD - E,), N, jnp.int32)
  srcp = jnp.concatenate([src, pad]).reshape(NS * NB_TILE, B)
  dstp = jnp.concatenate([dst, pad]).reshape(NS * NB_TILE, B)

  xp = jnp.zeros((NPAD, D), jnp.float32).at[:N].set(x)
  zr = jnp.zeros((ROWS_PER_TILE, DH), jnp.float32)
  zv = jnp.zeros((ROWS_PER_TILE,), jnp.float32)

  a1s_v = a1s.reshape(1, D)
  a1d_v = a1d.reshape(1, D)
  a2s_v = a2s.reshape(1, D)
  a2d_v = a2d.reshape(1, D)

  sc_edge = _make_sc_edge_kernel()

  xs1, asrc1, ad1, skip1 = _tc_prep(
      xp, W1s, W1d, a1s_v, a1d_v, Wl1, bl1.reshape(1, D))
  u1 = _tc_u(asrc1, ad1)
  numer1, denom1 = sc_edge(
      xs1, asrc1.reshape(NPAD), ad1.reshape(NPAD), u1.reshape(NPAD),
      srcp, dstp, zr, zv)

  xs2, asrc2, ad2, skip2 = _tc_mid(
      numer1, denom1[0].reshape(NPAD, 1), b1.reshape(1, D), skip1,
      W2s, W2d, a2s_v, a2d_v, Wl2, bl2.reshape(1, D))
  u2 = _tc_u(asrc2, ad2)
  numer2, denom2 = sc_edge(
      xs2, asrc2.reshape(NPAD), ad2.reshape(NPAD), u2.reshape(NPAD),
      srcp, dstp, zr, zv)

  out = _tc_final(numer2, denom2[0].reshape(NPAD, 1), b2.reshape(1, D),
                  skip2)
  return out[:N]
